# Initial kernel scaffold; baseline (speedup 1.0000x reference)
import jax
import jax.numpy as jnp
from jax.experimental import pallas as pl


def kernel(x, params, edge_index, batch):
    raise NotImplementedError("write your pallas kernel here")


# speedup vs baseline: 1.1886x; 1.1886x over previous
import functools
import math

import jax
import jax.numpy as jnp
from jax.experimental import pallas as pl
from jax.experimental.pallas import tpu as pltpu

N_NODES = 10000
N_EDGES = 320000
D_IN = 128
HID = 64
N_CLASSES = 2
N_GRAPHS = 100
EPS = 1e-5

_SELU_ALPHA = 1.6732632423543772
_SELU_SCALE = 1.0507009873554805

_INTERPRET = False


def _selu(x):
    return _SELU_SCALE * jnp.where(x > 0, x, _SELU_ALPHA * (jnp.exp(x) - 1.0))


def _stats_body(x_ref, b_ref, s1_ref, s2_ref, cnt_ref):
    i = pl.program_id(0)
    x = x_ref[...]
    batch = b_ref[0, 0]
    onehot = (batch[:, None] == jax.lax.broadcasted_iota(jnp.int32, (1, N_GRAPHS), 1)).astype(jnp.float32)

    @pl.when(i == 0)
    def _init():
        s1_ref[...] = jnp.zeros_like(s1_ref)
        s2_ref[...] = jnp.zeros_like(s2_ref)
        cnt_ref[...] = jnp.zeros_like(cnt_ref)

    s1_ref[...] += jnp.dot(onehot.T, x, preferred_element_type=jnp.float32)
    s2_ref[...] += jnp.dot(onehot.T, x * x, preferred_element_type=jnp.float32)
    cnt_ref[...] += jnp.sum(onehot, axis=0)[:, None]


def _graph_stats(x, batch_3d, blk):
    n, d = x.shape
    grid = n // blk
    return pl.pallas_call(
        _stats_body,
        grid=(grid,),
        in_specs=[
            pl.BlockSpec((blk, d), lambda i: (i, 0)),
            pl.BlockSpec((1, 1, blk), lambda i: (i, 0, 0)),
        ],
        out_specs=[
            pl.BlockSpec((N_GRAPHS, d), lambda i: (0, 0)),
            pl.BlockSpec((N_GRAPHS, d), lambda i: (0, 0)),
            pl.BlockSpec((N_GRAPHS, 128), lambda i: (0, 0)),
        ],
        out_shape=[
            jax.ShapeDtypeStruct((N_GRAPHS, d), jnp.float32),
            jax.ShapeDtypeStruct((N_GRAPHS, d), jnp.float32),
            jax.ShapeDtypeStruct((N_GRAPHS, 128), jnp.float32),
        ],
        interpret=_INTERPRET,
    )(x, batch_3d)


def _norm_proj_body(x_ref, b_ref, mean_ref, var_ref, gw_ref, gb_ref, gms_ref,
                    wa_ref, ba_ref, wb_ref, a_ref, out_b_ref):
    x = x_ref[...]
    batch = b_ref[0, 0]
    onehot = (batch[:, None] == jax.lax.broadcasted_iota(jnp.int32, (1, N_GRAPHS), 1)).astype(jnp.float32)
    mean = jnp.dot(onehot, mean_ref[...], preferred_element_type=jnp.float32)
    var = jnp.dot(onehot, var_ref[...], preferred_element_type=jnp.float32)
    out = x - gms_ref[...] * mean
    nrm = gw_ref[...] * (out * jax.lax.rsqrt(var + EPS)) + gb_ref[...]
    a_ref[...] = jnp.dot(nrm, wa_ref[...], preferred_element_type=jnp.float32) + ba_ref[...]
    out_b_ref[...] = jnp.dot(nrm, wb_ref[...], preferred_element_type=jnp.float32)


def _norm_proj(x, batch_3d, mean, var, gn, wa_t, ba, wb_t, blk):
    n, d = x.shape
    grid = n // blk
    gw = gn['weight'].reshape(1, d)
    gb = gn['bias'].reshape(1, d)
    gms = gn['mean_scale'].reshape(1, d)
    return pl.pallas_call(
        _norm_proj_body,
        grid=(grid,),
        in_specs=[
            pl.BlockSpec((blk, d), lambda i: (i, 0)),
            pl.BlockSpec((1, 1, blk), lambda i: (i, 0, 0)),
            pl.BlockSpec((N_GRAPHS, d), lambda i: (0, 0)),
            pl.BlockSpec((N_GRAPHS, d), lambda i: (0, 0)),
            pl.BlockSpec((1, d), lambda i: (0, 0)),
            pl.BlockSpec((1, d), lambda i: (0, 0)),
            pl.BlockSpec((1, d), lambda i: (0, 0)),
            pl.BlockSpec((d, HID), lambda i: (0, 0)),
            pl.BlockSpec((1, HID), lambda i: (0, 0)),
            pl.BlockSpec((d, HID), lambda i: (0, 0)),
        ],
        out_specs=[
            pl.BlockSpec((blk, HID), lambda i: (i, 0)),
            pl.BlockSpec((blk, HID), lambda i: (i, 0)),
        ],
        out_shape=[
            jax.ShapeDtypeStruct((n, HID), jnp.float32),
            jax.ShapeDtypeStruct((n, HID), jnp.float32),
        ],
        interpret=_INTERPRET,
    )(x, batch_3d, mean, var, gw, gb, gms, wa_t, ba.reshape(1, HID), wb_t)


def _edge_mlp_body(e_ref, w2_ref, b2_ref, w3_ref, b3_ref, out_ref):
    m = _selu(e_ref[...])
    m = _selu(jnp.dot(m, w2_ref[...], preferred_element_type=jnp.float32) + b2_ref[...])
    out_ref[...] = jnp.dot(m, w3_ref[...], preferred_element_type=jnp.float32) + b3_ref[...]


def _edge_mlp(epre, w2_t, b2, w3_t, b3, blk):
    e = epre.shape[0]
    grid = e // blk
    return pl.pallas_call(
        _edge_mlp_body,
        grid=(grid,),
        in_specs=[
            pl.BlockSpec((blk, HID), lambda i: (i, 0)),
            pl.BlockSpec((HID, HID), lambda i: (0, 0)),
            pl.BlockSpec((1, HID), lambda i: (0, 0)),
            pl.BlockSpec((HID, HID), lambda i: (0, 0)),
            pl.BlockSpec((1, HID), lambda i: (0, 0)),
        ],
        out_specs=pl.BlockSpec((blk, HID), lambda i: (i, 0)),
        out_shape=jax.ShapeDtypeStruct((e, HID), jnp.float32),
        interpret=_INTERPRET,
    )(epre, w2_t, b2.reshape(1, HID), w3_t, b3.reshape(1, HID))


def _head_body(s1_ref, cnt_ref, wd_ref, bd_ref, wo_ref, bo_ref, out_ref):
    cnt = jnp.maximum(cnt_ref[...][:, :1], 1.0)
    pooled = s1_ref[...] / cnt
    h = jnp.maximum(jnp.dot(pooled, wd_ref[...], preferred_element_type=jnp.float32) + bd_ref[...], 0.0)
    logits = jnp.dot(h, wo_ref[...], preferred_element_type=jnp.float32) + bo_ref[...]
    mx = jnp.max(logits, axis=1, keepdims=True)
    ex = jnp.exp(logits - mx)
    out_ref[...] = ex / jnp.sum(ex, axis=1, keepdims=True)


def _head(s1, cnt, dense, output):
    wd_t = dense['W'].T
    wo_t = output['W'].T
    return pl.pallas_call(
        _head_body,
        out_shape=jax.ShapeDtypeStruct((N_GRAPHS, N_CLASSES), jnp.float32),
        interpret=_INTERPRET,
    )(s1, cnt, wd_t, dense['b'].reshape(1, HID), wo_t, output['b'].reshape(1, N_CLASSES))


def _conv_block(h, batch_3d, src, dst, gn, mlp, blk):
    d = h.shape[1]
    s1, s2, cnt = _graph_stats(h, batch_3d, blk)
    cnt1 = jnp.maximum(cnt[:, :1], 1.0)
    mean = s1 / cnt1
    ms = gn['mean_scale'][None, :]
    var = s2 / cnt1 - (2.0 * ms - ms * ms) * mean * mean
    w1 = mlp['W1']
    w1a = w1[:, :d]
    w1b = w1[:, d:]
    wa_t = (w1a - w1b).T
    wb_t = w1b.T
    a, b = _norm_proj(h, batch_3d, mean, var, gn, wa_t, mlp['b1'], wb_t, blk)
    epre = a[dst] + b[src]
    m = _edge_mlp(epre, mlp['W2'].T, mlp['b2'], mlp['W3'].T, mlp['b3'], 2560)
    agg = jax.ops.segment_max(m, dst, num_segments=N_NODES)
    agg = jnp.maximum(agg, 0.0)
    return agg


def kernel(x, params, edge_index, batch):
    src = edge_index[0]
    dst = edge_index[1]
    batch_3d = batch.reshape(N_NODES // 2000, 1, 2000)

    h = _conv_block(x, batch_3d, src, dst, params['gn0'], params['conv1'], 2000)
    h = _conv_block(h, batch_3d, src, dst, params['gn1'], params['conv2'], 2000)
    h = _conv_block(h, batch_3d, src, dst, params['gn2'], params['conv3'], 2000)

    s1, _, cnt = _graph_stats(h, batch_3d, 2000)
    return _head(s1, cnt, params['dense'], params['output'])


# speedup vs baseline: 2.0780x; 1.7482x over previous
import functools
import math

import jax
import jax.numpy as jnp
from jax import lax
from jax.experimental import pallas as pl
from jax.experimental.pallas import tpu as pltpu
from jax.experimental.pallas import tpu_sc as plsc

N_NODES = 10000
N_EDGES = 320000
D_IN = 128
HID = 64
N_CLASSES = 2
N_GRAPHS = 100
EPS = 1e-5

_SELU_ALPHA = 1.6732632423543772
_SELU_SCALE = 1.0507009873554805

_INTERPRET = False


def _selu(x):
    return _SELU_SCALE * jnp.where(x > 0, x, _SELU_ALPHA * (jnp.exp(x) - 1.0))


def _stats_body(x_ref, b_ref, s1_ref, s2_ref, cnt_ref):
    i = pl.program_id(0)
    x = x_ref[...]
    batch = b_ref[0, 0]
    onehot = (batch[:, None] == jax.lax.broadcasted_iota(jnp.int32, (1, N_GRAPHS), 1)).astype(jnp.float32)

    @pl.when(i == 0)
    def _init():
        s1_ref[...] = jnp.zeros_like(s1_ref)
        s2_ref[...] = jnp.zeros_like(s2_ref)
        cnt_ref[...] = jnp.zeros_like(cnt_ref)

    s1_ref[...] += jnp.dot(onehot.T, x, preferred_element_type=jnp.float32)
    s2_ref[...] += jnp.dot(onehot.T, x * x, preferred_element_type=jnp.float32)
    cnt_ref[...] += jnp.sum(onehot, axis=0)[:, None]


def _graph_stats(x, batch_3d, blk):
    n, d = x.shape
    grid = n // blk
    return pl.pallas_call(
        _stats_body,
        grid=(grid,),
        in_specs=[
            pl.BlockSpec((blk, d), lambda i: (i, 0)),
            pl.BlockSpec((1, 1, blk), lambda i: (i, 0, 0)),
        ],
        out_specs=[
            pl.BlockSpec((N_GRAPHS, d), lambda i: (0, 0)),
            pl.BlockSpec((N_GRAPHS, d), lambda i: (0, 0)),
            pl.BlockSpec((N_GRAPHS, 128), lambda i: (0, 0)),
        ],
        out_shape=[
            jax.ShapeDtypeStruct((N_GRAPHS, d), jnp.float32),
            jax.ShapeDtypeStruct((N_GRAPHS, d), jnp.float32),
            jax.ShapeDtypeStruct((N_GRAPHS, 128), jnp.float32),
        ],
        interpret=_INTERPRET,
    )(x, batch_3d)


def _norm_proj_body(x_ref, b_ref, mean_ref, var_ref, gw_ref, gb_ref, gms_ref,
                    wcat_ref, bcat_ref, t_ref):
    x = x_ref[...]
    batch = b_ref[0, 0]
    onehot = (batch[:, None] == jax.lax.broadcasted_iota(jnp.int32, (1, N_GRAPHS), 1)).astype(jnp.float32)
    mean = jnp.dot(onehot, mean_ref[...], preferred_element_type=jnp.float32)
    var = jnp.dot(onehot, var_ref[...], preferred_element_type=jnp.float32)
    out = x - gms_ref[...] * mean
    nrm = gw_ref[...] * (out * jax.lax.rsqrt(var + EPS)) + gb_ref[...]
    t_ref[...] = jnp.dot(nrm, wcat_ref[...], preferred_element_type=jnp.float32) + bcat_ref[...]


def _norm_proj(x, batch_3d, mean, var, gn, wcat, bcat, blk):
    n, d = x.shape
    grid = n // blk
    gw = gn['weight'].reshape(1, d)
    gb = gn['bias'].reshape(1, d)
    gms = gn['mean_scale'].reshape(1, d)
    return pl.pallas_call(
        _norm_proj_body,
        grid=(grid,),
        in_specs=[
            pl.BlockSpec((blk, d), lambda i: (i, 0)),
            pl.BlockSpec((1, 1, blk), lambda i: (i, 0, 0)),
            pl.BlockSpec((N_GRAPHS, d), lambda i: (0, 0)),
            pl.BlockSpec((N_GRAPHS, d), lambda i: (0, 0)),
            pl.BlockSpec((1, d), lambda i: (0, 0)),
            pl.BlockSpec((1, d), lambda i: (0, 0)),
            pl.BlockSpec((1, d), lambda i: (0, 0)),
            pl.BlockSpec((d, 2 * HID), lambda i: (0, 0)),
            pl.BlockSpec((1, 2 * HID), lambda i: (0, 0)),
        ],
        out_specs=pl.BlockSpec((blk, 2 * HID), lambda i: (i, 0)),
        out_shape=jax.ShapeDtypeStruct((n, 2 * HID), jnp.float32),
        interpret=_INTERPRET,
    )(x, batch_3d, mean, var, gw, gb, gms, wcat, bcat)


def _edge_mlp_body(e_ref, w2_ref, b2_ref, w3_ref, b3_ref, out_ref):
    m = _selu(e_ref[...])
    m = _selu(jnp.dot(m, w2_ref[...], preferred_element_type=jnp.float32) + b2_ref[...])
    out_ref[...] = jnp.dot(m, w3_ref[...], preferred_element_type=jnp.float32) + b3_ref[...]


def _edge_mlp(epre, w2d, b2d, w3d, b3d, blk):
    e = epre.shape[0]
    grid = e // blk
    h2 = 2 * HID
    return pl.pallas_call(
        _edge_mlp_body,
        grid=(grid,),
        in_specs=[
            pl.BlockSpec((blk, h2), lambda i: (i, 0)),
            pl.BlockSpec((h2, h2), lambda i: (0, 0)),
            pl.BlockSpec((1, h2), lambda i: (0, 0)),
            pl.BlockSpec((h2, h2), lambda i: (0, 0)),
            pl.BlockSpec((1, h2), lambda i: (0, 0)),
        ],
        out_specs=pl.BlockSpec((blk, h2), lambda i: (i, 0)),
        out_shape=jax.ShapeDtypeStruct((e, h2), jnp.float32),
        interpret=_INTERPRET,
    )(epre, w2d, b2d, w3d, b3d)


def _head_body(s1_ref, cnt_ref, wd_ref, bd_ref, wo_ref, bo_ref, out_ref):
    cnt = jnp.maximum(cnt_ref[...][:, :1], 1.0)
    pooled = s1_ref[...] / cnt
    h = jnp.maximum(jnp.dot(pooled, wd_ref[...], preferred_element_type=jnp.float32) + bd_ref[...], 0.0)
    logits = jnp.dot(h, wo_ref[...], preferred_element_type=jnp.float32) + bo_ref[...]
    mx = jnp.max(logits, axis=1, keepdims=True)
    ex = jnp.exp(logits - mx)
    out_ref[...] = ex / jnp.sum(ex, axis=1, keepdims=True)


def _head(s1, cnt, dense, output):
    wd_t = dense['W'].T
    wo_t = output['W'].T
    return pl.pallas_call(
        _head_body,
        out_shape=jax.ShapeDtypeStruct((N_GRAPHS, N_CLASSES), jnp.float32),
        interpret=_INTERPRET,
    )(s1, cnt, wd_t, dense['b'].reshape(1, HID), wo_t, output['b'].reshape(1, N_CLASSES))


_SC_NC = 2
_SC_NS = 16
_NW = _SC_NC * _SC_NS
_EPW = N_EDGES // _NW
_ECHUNK = 400
_L = 16


def _sc_gather_body(t_hbm, src_hbm, dst_hbm, epre_hbm,
                    dstv, srcv, rowsd, rowss, outv, sem):
    wid = lax.axis_index("s") * _SC_NC + lax.axis_index("c")
    base0 = wid * _EPW

    def body(i, carry):
        base = base0 + i * _ECHUNK
        pltpu.sync_copy(dst_hbm.at[pl.ds(base, _ECHUNK)], dstv)
        pltpu.sync_copy(src_hbm.at[pl.ds(base, _ECHUNK)], srcv)
        cd = pltpu.async_copy(t_hbm.at[dstv], rowsd, sem)
        cs = pltpu.async_copy(t_hbm.at[srcv], rowss, sem)
        cd.wait()
        cs.wait()

        def pair(p, c2):
            for half in range(2):
                e = 2 * p + half
                for j in range(4):
                    outv[p, pl.ds(64 * half + 16 * j, _L)] = (
                        rowsd[e, pl.ds(16 * j, _L)] + rowss[e, pl.ds(64 + 16 * j, _L)])
            return c2

        lax.fori_loop(0, _ECHUNK // 2, pair, 0)
        obase = pl.multiple_of(base // 2, 8)
        pltpu.sync_copy(outv, epre_hbm.at[pl.ds(obase, _ECHUNK // 2)])
        return carry

    lax.fori_loop(0, _EPW // _ECHUNK, body, 0)


_sc_gather = functools.partial(
    pl.kernel,
    mesh=plsc.VectorSubcoreMesh(core_axis_name="c", subcore_axis_name="s"),
    out_type=jax.ShapeDtypeStruct((N_EDGES // 2, 2 * HID), jnp.float32),
    scratch_types=[
        pltpu.VMEM((_ECHUNK,), jnp.int32),
        pltpu.VMEM((_ECHUNK,), jnp.int32),
        pltpu.VMEM((_ECHUNK, 2 * HID), jnp.float32),
        pltpu.VMEM((_ECHUNK, 2 * HID), jnp.float32),
        pltpu.VMEM((_ECHUNK // 2, 2 * HID), jnp.float32),
        pltpu.SemaphoreType.DMA,
    ],
)(_sc_gather_body)


def _conv_block(h, batch_3d, src, dst, gn, mlp, blk):
    d = h.shape[1]
    s1, s2, cnt = _graph_stats(h, batch_3d, blk)
    cnt1 = jnp.maximum(cnt[:, :1], 1.0)
    mean = s1 / cnt1
    ms = gn['mean_scale'][None, :]
    var = s2 / cnt1 - (2.0 * ms - ms * ms) * mean * mean
    w1 = mlp['W1']
    w1a = w1[:, :d]
    w1b = w1[:, d:]
    wcat = jnp.concatenate([(w1a - w1b).T, w1b.T], axis=1)
    bcat = jnp.concatenate([mlp['b1'], jnp.zeros((HID,), jnp.float32)]).reshape(1, 2 * HID)
    t = _norm_proj(h, batch_3d, mean, var, gn, wcat, bcat, blk)
    epre = _sc_gather(t, src, dst)
    z = jnp.zeros((HID, HID), jnp.float32)
    w2d = jnp.block([[mlp['W2'].T, z], [z, mlp['W2'].T]])
    w3d = jnp.block([[mlp['W3'].T, z], [z, mlp['W3'].T]])
    b2d = jnp.tile(mlp['b2'], 2).reshape(1, 2 * HID)
    b3d = jnp.tile(mlp['b3'], 2).reshape(1, 2 * HID)
    m = _edge_mlp(epre, w2d, b2d, w3d, b3d, 1280)
    m64 = m.reshape(N_EDGES, HID)
    agg = jax.ops.segment_max(m64, dst, num_segments=N_NODES)
    agg = jnp.maximum(agg, 0.0)
    return agg


def kernel(x, params, edge_index, batch):
    src = edge_index[0]
    dst = edge_index[1]
    batch_3d = batch.reshape(N_NODES // 2000, 1, 2000)

    h = _conv_block(x, batch_3d, src, dst, params['gn0'], params['conv1'], 2000)
    h = _conv_block(h, batch_3d, src, dst, params['gn1'], params['conv2'], 2000)
    h = _conv_block(h, batch_3d, src, dst, params['gn2'], params['conv3'], 2000)

    s1, _, cnt = _graph_stats(h, batch_3d, 2000)
    return _head(s1, cnt, params['dense'], params['output'])


# speedup vs baseline: 2.0941x; 1.0078x over previous
import functools
import math

import jax
import jax.numpy as jnp
from jax import lax
from jax.experimental import pallas as pl
from jax.experimental.pallas import tpu as pltpu
from jax.experimental.pallas import tpu_sc as plsc

N_NODES = 10000
N_EDGES = 320000
D_IN = 128
HID = 64
N_CLASSES = 2
N_GRAPHS = 100
EPS = 1e-5

_SELU_ALPHA = 1.6732632423543772
_SELU_SCALE = 1.0507009873554805

_INTERPRET = False


def _selu(x):
    return _SELU_SCALE * jnp.where(x > 0, x, _SELU_ALPHA * (jnp.exp(x) - 1.0))


def _stats_body(x_ref, b_ref, s1_ref, s2_ref, cnt_ref):
    i = pl.program_id(0)
    x = x_ref[...]
    batch = b_ref[0, 0]
    onehot = (batch[:, None] == jax.lax.broadcasted_iota(jnp.int32, (1, N_GRAPHS), 1)).astype(jnp.float32)

    @pl.when(i == 0)
    def _init():
        s1_ref[...] = jnp.zeros_like(s1_ref)
        s2_ref[...] = jnp.zeros_like(s2_ref)
        cnt_ref[...] = jnp.zeros_like(cnt_ref)

    s1_ref[...] += jnp.dot(onehot.T, x, preferred_element_type=jnp.float32)
    s2_ref[...] += jnp.dot(onehot.T, x * x, preferred_element_type=jnp.float32)
    cnt_ref[...] += jnp.sum(onehot, axis=0)[:, None]


def _graph_stats(x, batch_3d, blk):
    n, d = x.shape
    grid = n // blk
    return pl.pallas_call(
        _stats_body,
        grid=(grid,),
        in_specs=[
            pl.BlockSpec((blk, d), lambda i: (i, 0)),
            pl.BlockSpec((1, 1, blk), lambda i: (i, 0, 0)),
        ],
        out_specs=[
            pl.BlockSpec((N_GRAPHS, d), lambda i: (0, 0)),
            pl.BlockSpec((N_GRAPHS, d), lambda i: (0, 0)),
            pl.BlockSpec((N_GRAPHS, 128), lambda i: (0, 0)),
        ],
        out_shape=[
            jax.ShapeDtypeStruct((N_GRAPHS, d), jnp.float32),
            jax.ShapeDtypeStruct((N_GRAPHS, d), jnp.float32),
            jax.ShapeDtypeStruct((N_GRAPHS, 128), jnp.float32),
        ],
        interpret=_INTERPRET,
    )(x, batch_3d)


def _norm_proj_body(x_ref, b_ref, mean_ref, var_ref, gw_ref, gb_ref, gms_ref,
                    wcat_ref, bcat_ref, t_ref):
    x = x_ref[...]
    batch = b_ref[0, 0]
    onehot = (batch[:, None] == jax.lax.broadcasted_iota(jnp.int32, (1, N_GRAPHS), 1)).astype(jnp.float32)
    mean = jnp.dot(onehot, mean_ref[...], preferred_element_type=jnp.float32)
    var = jnp.dot(onehot, var_ref[...], preferred_element_type=jnp.float32)
    out = x - gms_ref[...] * mean
    nrm = gw_ref[...] * (out * jax.lax.rsqrt(var + EPS)) + gb_ref[...]
    t_ref[...] = jnp.dot(nrm, wcat_ref[...], preferred_element_type=jnp.float32) + bcat_ref[...]


def _norm_proj(x, batch_3d, mean, var, gn, wcat, bcat, blk):
    n, d = x.shape
    grid = n // blk
    gw = gn['weight'].reshape(1, d)
    gb = gn['bias'].reshape(1, d)
    gms = gn['mean_scale'].reshape(1, d)
    return pl.pallas_call(
        _norm_proj_body,
        grid=(grid,),
        in_specs=[
            pl.BlockSpec((blk, d), lambda i: (i, 0)),
            pl.BlockSpec((1, 1, blk), lambda i: (i, 0, 0)),
            pl.BlockSpec((N_GRAPHS, d), lambda i: (0, 0)),
            pl.BlockSpec((N_GRAPHS, d), lambda i: (0, 0)),
            pl.BlockSpec((1, d), lambda i: (0, 0)),
            pl.BlockSpec((1, d), lambda i: (0, 0)),
            pl.BlockSpec((1, d), lambda i: (0, 0)),
            pl.BlockSpec((d, 2 * HID), lambda i: (0, 0)),
            pl.BlockSpec((1, 2 * HID), lambda i: (0, 0)),
        ],
        out_specs=pl.BlockSpec((blk, 2 * HID), lambda i: (i, 0)),
        out_shape=jax.ShapeDtypeStruct((n, 2 * HID), jnp.float32),
        interpret=_INTERPRET,
    )(x, batch_3d, mean, var, gw, gb, gms, wcat, bcat)


def _edge_mlp_body(e_ref, w2_ref, b2_ref, w3_ref, b3_ref, out_ref):
    m = _selu(e_ref[...])
    m = _selu(jnp.dot(m, w2_ref[...], preferred_element_type=jnp.float32) + b2_ref[...])
    out_ref[...] = jnp.dot(m, w3_ref[...], preferred_element_type=jnp.float32) + b3_ref[...]


def _edge_mlp(epre, w2d, b2d, w3d, b3d, blk):
    e = epre.shape[0]
    grid = e // blk
    h2 = 2 * HID
    return pl.pallas_call(
        _edge_mlp_body,
        grid=(grid,),
        in_specs=[
            pl.BlockSpec((blk, h2), lambda i: (i, 0)),
            pl.BlockSpec((h2, h2), lambda i: (0, 0)),
            pl.BlockSpec((1, h2), lambda i: (0, 0)),
            pl.BlockSpec((h2, h2), lambda i: (0, 0)),
            pl.BlockSpec((1, h2), lambda i: (0, 0)),
        ],
        out_specs=pl.BlockSpec((blk, h2), lambda i: (i, 0)),
        out_shape=jax.ShapeDtypeStruct((e, h2), jnp.float32),
        interpret=_INTERPRET,
    )(epre, w2d, b2d, w3d, b3d)


def _head_body(s1_ref, cnt_ref, wd_ref, bd_ref, wo_ref, bo_ref, out_ref):
    cnt = jnp.maximum(cnt_ref[...][:, :1], 1.0)
    pooled = s1_ref[...] / cnt
    h = jnp.maximum(jnp.dot(pooled, wd_ref[...], preferred_element_type=jnp.float32) + bd_ref[...], 0.0)
    logits = jnp.dot(h, wo_ref[...], preferred_element_type=jnp.float32) + bo_ref[...]
    mx = jnp.max(logits, axis=1, keepdims=True)
    ex = jnp.exp(logits - mx)
    out_ref[...] = ex / jnp.sum(ex, axis=1, keepdims=True)


def _head(s1, cnt, dense, output):
    wd_t = dense['W'].T
    wo_t = output['W'].T
    return pl.pallas_call(
        _head_body,
        out_shape=jax.ShapeDtypeStruct((N_GRAPHS, N_CLASSES), jnp.float32),
        interpret=_INTERPRET,
    )(s1, cnt, wd_t, dense['b'].reshape(1, HID), wo_t, output['b'].reshape(1, N_CLASSES))


_SC_NC = 2
_SC_NS = 16
_NW = _SC_NC * _SC_NS
_EPW = N_EDGES // _NW
_ECHUNK = 400
_L = 16


def _sc_gather_body(t_hbm, src_hbm, dst_hbm, epre_hbm,
                    dstv, srcv, rowsd, rowss, outv, sem):
    wid = lax.axis_index("s") * _SC_NC + lax.axis_index("c")
    base0 = wid * _EPW

    def body(i, carry):
        base = base0 + i * _ECHUNK
        pltpu.sync_copy(dst_hbm.at[pl.ds(base, _ECHUNK)], dstv)
        pltpu.sync_copy(src_hbm.at[pl.ds(base, _ECHUNK)], srcv)
        cd = pltpu.async_copy(t_hbm.at[dstv], rowsd, sem)
        cs = pltpu.async_copy(t_hbm.at[srcv], rowss, sem)
        cd.wait()
        cs.wait()

        def pair(p, c2):
            for half in range(2):
                e = 2 * p + half
                for j in range(4):
                    outv[p, pl.ds(64 * half + 16 * j, _L)] = (
                        rowsd[e, pl.ds(16 * j, _L)] + rowss[e, pl.ds(64 + 16 * j, _L)])
            return c2

        lax.fori_loop(0, _ECHUNK // 2, pair, 0)
        obase = pl.multiple_of(base // 2, 8)
        pltpu.sync_copy(outv, epre_hbm.at[pl.ds(obase, _ECHUNK // 2)])
        return carry

    lax.fori_loop(0, _EPW // _ECHUNK, body, 0)


_sc_gather = functools.partial(
    pl.kernel,
    mesh=plsc.VectorSubcoreMesh(core_axis_name="c", subcore_axis_name="s"),
    out_type=jax.ShapeDtypeStruct((N_EDGES // 2, 2 * HID), jnp.float32),
    scratch_types=[
        pltpu.VMEM((_ECHUNK,), jnp.int32),
        pltpu.VMEM((_ECHUNK,), jnp.int32),
        pltpu.VMEM((_ECHUNK, 2 * HID), jnp.float32),
        pltpu.VMEM((_ECHUNK, 2 * HID), jnp.float32),
        pltpu.VMEM((_ECHUNK // 2, 2 * HID), jnp.float32),
        pltpu.SemaphoreType.DMA,
    ],
)(_sc_gather_body)


_NPT = 320
_PCAP = N_EDGES + 2048
_PBLK = 2048
_DCH = 2048
_SCH = 512


def _iota16():
    return lax.iota(jnp.int32, 16)


def _sc_pack_body(dst_hbm, p_hbm, c_hbm, dv, buf, cv, sem):
    wid = lax.axis_index("s") * _SC_NC + lax.axis_index("c")
    lo = wid * _NPT
    pbase = wid * _PCAP
    iota = _iota16()

    def chunk_body(ci, carry):
        pos, fl, tot_vec = carry
        cb = ci * _DCH
        pltpu.sync_copy(dst_hbm.at[pl.ds(cb, _DCH)], dv)

        def sub_body(s, c2):
            posv, totv = c2
            d = dv[pl.ds(s * 16, 16)]
            dl = d - lo
            mask = (dl >= 0) & (dl < _NPT)
            eid = cb + s * 16 + iota
            packed = (eid << 9) | jnp.where(mask, dl, 0)
            _, sortedv, _ = plsc.sort_key_val(iota, packed, mask=mask)
            plsc.store_scatter(buf, [posv + iota], sortedv)
            c16 = plsc.all_reduce_population_count(mask)
            return posv + c16, totv + c16

        posv0 = jnp.full((16,), pos, jnp.int32)
        posv, tot_vec = lax.fori_loop(0, _DCH // 16, sub_body, (posv0, tot_vec))
        pos = posv[0]

        def flush(args):
            pos3, fl3 = args
            dst_off = pl.multiple_of(pbase + fl3 * _PBLK, 8)
            pltpu.sync_copy(buf.at[pl.ds(0, _PBLK)], p_hbm.at[pl.ds(dst_off, _PBLK)])

            def mv(r, c4):
                buf[pl.ds(r * 16, 16)] = buf[pl.ds(_PBLK + r * 16, 16)]
                return c4

            lax.fori_loop(0, _PBLK // 16, mv, 0)
            return pos3 - _PBLK, fl3 + 1

        pos, fl = lax.cond(pos >= _PBLK, flush, lambda a: a, (pos, fl))
        return pos, fl, tot_vec

    zero = jnp.zeros((), jnp.int32)
    pos, fl, tot_vec = lax.fori_loop(
        0, N_EDGES // _DCH, chunk_body, (zero, zero, jnp.zeros((16,), jnp.int32)))
    dst_off = pl.multiple_of(pbase + fl * _PBLK, 8)
    pltpu.sync_copy(buf.at[pl.ds(0, _PBLK)], p_hbm.at[pl.ds(dst_off, _PBLK)])
    cv[...] = tot_vec
    pltpu.sync_copy(cv, c_hbm.at[pl.ds(wid * 16, 16)])


_sc_pack = functools.partial(
    pl.kernel,
    mesh=plsc.VectorSubcoreMesh(core_axis_name="c", subcore_axis_name="s"),
    compiler_params=pltpu.CompilerParams(needs_layout_passes=False),
    out_type=[
        jax.ShapeDtypeStruct((_NW * _PCAP,), jnp.int32),
        jax.ShapeDtypeStruct((_NW * 16,), jnp.int32),
    ],
    scratch_types=[
        pltpu.VMEM((_DCH,), jnp.int32),
        pltpu.VMEM((2 * _PBLK + 16,), jnp.int32),
        pltpu.VMEM((16,), jnp.int32),
        pltpu.SemaphoreType.DMA,
    ],
)(_sc_pack_body)


def _sc_scatter_body(p_hbm, c_hbm, m_hbm, agg_hbm,
                     packed_v, ridx, rows_v, cv, acc, sem):
    wid = lax.axis_index("s") * _SC_NC + lax.axis_index("c")
    iota = _iota16()

    def zero(r, c):
        for j in range(8):
            acc[r, pl.ds(16 * j, 16)] = jnp.zeros((16,), jnp.float32)
        return c

    lax.fori_loop(0, _NPT // 2, zero, 0)

    pltpu.sync_copy(c_hbm.at[pl.ds(wid * 16, 16)], cv)
    cnt = cv[...][0]
    nchunks = (cnt + _SCH - 1) // _SCH

    def chunk(k, carry):
        base = pl.multiple_of(wid * _PCAP + k * _SCH, 8)
        pltpu.sync_copy(p_hbm.at[pl.ds(base, _SCH)], packed_v)

        def mkidx(s, c2):
            pk = packed_v[pl.ds(s * 16, 16)]
            valid = (k * _SCH + s * 16 + iota) < cnt
            ridx[pl.ds(s * 16, 16)] = jnp.where(valid, pk >> 10, 0)
            return c2

        lax.fori_loop(0, _SCH // 16, mkidx, 0)
        pltpu.async_copy(m_hbm.at[ridx], rows_v, sem).wait()
        nedge = jnp.minimum(_SCH, cnt - k * _SCH)

        def edge(i, c3):
            isp = jnp.full((16,), i, jnp.int32)
            pk = plsc.load_gather(packed_v, [isp])
            dl = pk & 511
            par = (pk >> 9) & 1
            arow = dl >> 1
            acol0 = (dl & 1) * 64
            mcol0 = par * 64
            for j in range(4):
                mcol = mcol0 + 16 * j + iota
                acol = acol0 + 16 * j + iota
                mv = plsc.load_gather(rows_v, [isp, mcol])
                av = plsc.load_gather(acc, [arow, acol])
                plsc.store_scatter(acc, [arow, acol], jnp.maximum(av, mv))
            return c3

        lax.fori_loop(0, nedge, edge, 0)
        return carry

    lax.fori_loop(0, nchunks, chunk, 0)
    obase = pl.multiple_of(wid * (_NPT // 2), 8)
    pltpu.sync_copy(acc, agg_hbm.at[pl.ds(obase, _NPT // 2)])


_sc_scatter = functools.partial(
    pl.kernel,
    mesh=plsc.VectorSubcoreMesh(core_axis_name="c", subcore_axis_name="s"),
    compiler_params=pltpu.CompilerParams(needs_layout_passes=False),
    out_type=jax.ShapeDtypeStruct((_NW * _NPT // 2, 2 * HID), jnp.float32),
    scratch_types=[
        pltpu.VMEM((_SCH,), jnp.int32),
        pltpu.VMEM((_SCH,), jnp.int32),
        pltpu.VMEM((_SCH, 2 * HID), jnp.float32),
        pltpu.VMEM((16,), jnp.int32),
        pltpu.VMEM((_NPT // 2, 2 * HID), jnp.float32),
        pltpu.SemaphoreType.DMA,
    ],
)(_sc_scatter_body)


def _conv_block(h, batch_3d, src, dst, p_arr, c_arr, gn, mlp, blk):
    d = h.shape[1]
    s1, s2, cnt = _graph_stats(h, batch_3d, blk)
    cnt1 = jnp.maximum(cnt[:, :1], 1.0)
    mean = s1 / cnt1
    ms = gn['mean_scale'][None, :]
    var = s2 / cnt1 - (2.0 * ms - ms * ms) * mean * mean
    w1 = mlp['W1']
    w1a = w1[:, :d]
    w1b = w1[:, d:]
    wcat = jnp.concatenate([(w1a - w1b).T, w1b.T], axis=1)
    bcat = jnp.concatenate([mlp['b1'], jnp.zeros((HID,), jnp.float32)]).reshape(1, 2 * HID)
    t = _norm_proj(h, batch_3d, mean, var, gn, wcat, bcat, blk)
    epre = _sc_gather(t, src, dst)
    z = jnp.zeros((HID, HID), jnp.float32)
    w2d = jnp.block([[mlp['W2'].T, z], [z, mlp['W2'].T]])
    w3d = jnp.block([[mlp['W3'].T, z], [z, mlp['W3'].T]])
    b2d = jnp.tile(mlp['b2'], 2).reshape(1, 2 * HID)
    b3d = jnp.tile(mlp['b3'], 2).reshape(1, 2 * HID)
    m = _edge_mlp(epre, w2d, b2d, w3d, b3d, 1280)
    agg = _sc_scatter(p_arr, c_arr, m)
    return agg.reshape(_NW * _NPT, HID)[:N_NODES]


def kernel(x, params, edge_index, batch):
    src = edge_index[0]
    dst = edge_index[1]
    batch_3d = batch.reshape(N_NODES // 2000, 1, 2000)
    p_arr, c_arr = _sc_pack(dst)

    h = _conv_block(x, batch_3d, src, dst, p_arr, c_arr, params['gn0'], params['conv1'], 2000)
    h = _conv_block(h, batch_3d, src, dst, p_arr, c_arr, params['gn1'], params['conv2'], 2000)
    h = _conv_block(h, batch_3d, src, dst, p_arr, c_arr, params['gn2'], params['conv3'], 2000)

    s1, _, cnt = _graph_stats(h, batch_3d, 2000)
    return _head(s1, cnt, params['dense'], params['output'])


# speedup vs baseline: 2.2703x; 1.0841x over previous
import functools
import math

import jax
import jax.numpy as jnp
from jax import lax
from jax.experimental import pallas as pl
from jax.experimental.pallas import tpu as pltpu
from jax.experimental.pallas import tpu_sc as plsc

N_NODES = 10000
N_EDGES = 320000
D_IN = 128
HID = 64
N_CLASSES = 2
N_GRAPHS = 100
EPS = 1e-5

_SELU_ALPHA = 1.6732632423543772
_SELU_SCALE = 1.0507009873554805

_INTERPRET = False


def _selu(x):
    return _SELU_SCALE * jnp.where(x > 0, x, _SELU_ALPHA * (jnp.exp(x) - 1.0))


def _stats_body(x_ref, b_ref, s1_ref, s2_ref, cnt_ref):
    i = pl.program_id(0)
    x = x_ref[...]
    batch = b_ref[0, 0]
    onehot = (batch[:, None] == jax.lax.broadcasted_iota(jnp.int32, (1, N_GRAPHS), 1)).astype(jnp.float32)

    @pl.when(i == 0)
    def _init():
        s1_ref[...] = jnp.zeros_like(s1_ref)
        s2_ref[...] = jnp.zeros_like(s2_ref)
        cnt_ref[...] = jnp.zeros_like(cnt_ref)

    s1_ref[...] += jnp.dot(onehot.T, x, preferred_element_type=jnp.float32)
    s2_ref[...] += jnp.dot(onehot.T, x * x, preferred_element_type=jnp.float32)
    cnt_ref[...] += jnp.sum(onehot, axis=0)[:, None]


def _graph_stats(x, batch_3d, blk):
    n, d = x.shape
    grid = n // blk
    return pl.pallas_call(
        _stats_body,
        grid=(grid,),
        in_specs=[
            pl.BlockSpec((blk, d), lambda i: (i, 0)),
            pl.BlockSpec((1, 1, blk), lambda i: (i, 0, 0)),
        ],
        out_specs=[
            pl.BlockSpec((N_GRAPHS, d), lambda i: (0, 0)),
            pl.BlockSpec((N_GRAPHS, d), lambda i: (0, 0)),
            pl.BlockSpec((N_GRAPHS, 128), lambda i: (0, 0)),
        ],
        out_shape=[
            jax.ShapeDtypeStruct((N_GRAPHS, d), jnp.float32),
            jax.ShapeDtypeStruct((N_GRAPHS, d), jnp.float32),
            jax.ShapeDtypeStruct((N_GRAPHS, 128), jnp.float32),
        ],
        interpret=_INTERPRET,
    )(x, batch_3d)


def _norm_proj_body(x_ref, b_ref, mean_ref, var_ref, gw_ref, gb_ref, gms_ref,
                    wcat_ref, bcat_ref, t_ref):
    x = x_ref[...]
    batch = b_ref[0, 0]
    onehot = (batch[:, None] == jax.lax.broadcasted_iota(jnp.int32, (1, N_GRAPHS), 1)).astype(jnp.float32)
    mean = jnp.dot(onehot, mean_ref[...], preferred_element_type=jnp.float32)
    var = jnp.dot(onehot, var_ref[...], preferred_element_type=jnp.float32)
    out = x - gms_ref[...] * mean
    nrm = gw_ref[...] * (out * jax.lax.rsqrt(var + EPS)) + gb_ref[...]
    t_ref[...] = jnp.dot(nrm, wcat_ref[...], preferred_element_type=jnp.float32) + bcat_ref[...]


def _norm_proj(x, batch_3d, mean, var, gn, wcat, bcat, blk):
    n, d = x.shape
    grid = n // blk
    gw = gn['weight'].reshape(1, d)
    gb = gn['bias'].reshape(1, d)
    gms = gn['mean_scale'].reshape(1, d)
    return pl.pallas_call(
        _norm_proj_body,
        grid=(grid,),
        in_specs=[
            pl.BlockSpec((blk, d), lambda i: (i, 0)),
            pl.BlockSpec((1, 1, blk), lambda i: (i, 0, 0)),
            pl.BlockSpec((N_GRAPHS, d), lambda i: (0, 0)),
            pl.BlockSpec((N_GRAPHS, d), lambda i: (0, 0)),
            pl.BlockSpec((1, d), lambda i: (0, 0)),
            pl.BlockSpec((1, d), lambda i: (0, 0)),
            pl.BlockSpec((1, d), lambda i: (0, 0)),
            pl.BlockSpec((d, 2 * HID), lambda i: (0, 0)),
            pl.BlockSpec((1, 2 * HID), lambda i: (0, 0)),
        ],
        out_specs=pl.BlockSpec((blk, 2 * HID), lambda i: (i, 0)),
        out_shape=jax.ShapeDtypeStruct((n, 2 * HID), jnp.float32),
        interpret=_INTERPRET,
    )(x, batch_3d, mean, var, gw, gb, gms, wcat, bcat)


def _edge_mlp_body(gd1_ref, gs1_ref, gd2_ref, gs2_ref,
                   w2_ref, b2_ref, w3_ref, b3_ref, out_ref):
    e1 = gd1_ref[...][:, :HID] + gs1_ref[...][:, HID:]
    e2 = gd2_ref[...][:, :HID] + gs2_ref[...][:, HID:]
    m = _selu(jnp.concatenate([e1, e2], axis=1))
    m = _selu(jnp.dot(m, w2_ref[...], preferred_element_type=jnp.float32) + b2_ref[...])
    out_ref[...] = jnp.dot(m, w3_ref[...], preferred_element_type=jnp.float32) + b3_ref[...]


def _edge_mlp(gd, gs, w2d, b2d, w3d, b3d, blk):
    h2 = 2 * HID
    half_blocks = (N_EDGES // 2) // blk
    grid = half_blocks
    return pl.pallas_call(
        _edge_mlp_body,
        grid=(grid,),
        in_specs=[
            pl.BlockSpec((blk, h2), lambda i: (i, 0)),
            pl.BlockSpec((blk, h2), lambda i: (i, 0)),
            pl.BlockSpec((blk, h2), lambda i: (i + half_blocks, 0)),
            pl.BlockSpec((blk, h2), lambda i: (i + half_blocks, 0)),
            pl.BlockSpec((h2, h2), lambda i: (0, 0)),
            pl.BlockSpec((1, h2), lambda i: (0, 0)),
            pl.BlockSpec((h2, h2), lambda i: (0, 0)),
            pl.BlockSpec((1, h2), lambda i: (0, 0)),
        ],
        out_specs=pl.BlockSpec((blk, h2), lambda i: (i, 0)),
        out_shape=jax.ShapeDtypeStruct((N_EDGES // 2, h2), jnp.float32),
        interpret=_INTERPRET,
    )(gd, gs, gd, gs, w2d, b2d, w3d, b3d)


def _head_body(s1_ref, cnt_ref, wd_ref, bd_ref, wo_ref, bo_ref, out_ref):
    cnt = jnp.maximum(cnt_ref[...][:, :1], 1.0)
    pooled = s1_ref[...] / cnt
    h = jnp.maximum(jnp.dot(pooled, wd_ref[...], preferred_element_type=jnp.float32) + bd_ref[...], 0.0)
    logits = jnp.dot(h, wo_ref[...], preferred_element_type=jnp.float32) + bo_ref[...]
    mx = jnp.max(logits, axis=1, keepdims=True)
    ex = jnp.exp(logits - mx)
    out_ref[...] = ex / jnp.sum(ex, axis=1, keepdims=True)


def _head(s1, cnt, dense, output):
    wd_t = dense['W'].T
    wo_t = output['W'].T
    return pl.pallas_call(
        _head_body,
        out_shape=jax.ShapeDtypeStruct((N_GRAPHS, N_CLASSES), jnp.float32),
        interpret=_INTERPRET,
    )(s1, cnt, wd_t, dense['b'].reshape(1, HID), wo_t, output['b'].reshape(1, N_CLASSES))


_SC_NC = 2
_SC_NS = 16
_NW = _SC_NC * _SC_NS
_EPW = N_EDGES // _NW
_ECHUNK = 400
_L = 16


def _sc_gather_body(t_hbm, src_hbm, dst_hbm, gd_hbm, gs_hbm,
                    dstv, srcv, rowsd, rowss, sem):
    wid = lax.axis_index("s") * _SC_NC + lax.axis_index("c")
    base0 = wid * _EPW

    def body(i, carry):
        base = pl.multiple_of(base0 + i * _ECHUNK, 8)
        ci = pltpu.async_copy(dst_hbm.at[pl.ds(base, _ECHUNK)], dstv, sem)
        cj = pltpu.async_copy(src_hbm.at[pl.ds(base, _ECHUNK)], srcv, sem)
        ci.wait()
        cj.wait()
        cd = pltpu.async_copy(t_hbm.at[dstv], rowsd, sem)
        cs = pltpu.async_copy(t_hbm.at[srcv], rowss, sem)
        cd.wait()
        cs.wait()
        co = pltpu.async_copy(rowsd, gd_hbm.at[pl.ds(base, _ECHUNK)], sem)
        cp = pltpu.async_copy(rowss, gs_hbm.at[pl.ds(base, _ECHUNK)], sem)
        co.wait()
        cp.wait()
        return carry

    lax.fori_loop(0, _EPW // _ECHUNK, body, 0)


_sc_gather = functools.partial(
    pl.kernel,
    mesh=plsc.VectorSubcoreMesh(core_axis_name="c", subcore_axis_name="s"),
    out_type=[
        jax.ShapeDtypeStruct((N_EDGES, 2 * HID), jnp.float32),
        jax.ShapeDtypeStruct((N_EDGES, 2 * HID), jnp.float32),
    ],
    scratch_types=[
        pltpu.VMEM((_ECHUNK,), jnp.int32),
        pltpu.VMEM((_ECHUNK,), jnp.int32),
        pltpu.VMEM((_ECHUNK, 2 * HID), jnp.float32),
        pltpu.VMEM((_ECHUNK, 2 * HID), jnp.float32),
        pltpu.SemaphoreType.DMA,
    ],
)(_sc_gather_body)


_NPT = 320
_PCAP = N_EDGES + 2048
_PBLK = 2048
_DCH = 2048
_SCH = 512


def _iota16():
    return lax.iota(jnp.int32, 16)


def _sc_pack_body(dst_hbm, p_hbm, c_hbm, dv, buf, cv, sem):
    wid = lax.axis_index("s") * _SC_NC + lax.axis_index("c")
    lo = wid * _NPT
    pbase = wid * _PCAP
    iota = _iota16()

    def chunk_body(ci, carry):
        pos, fl, tot_vec = carry
        cb = ci * _DCH
        pltpu.sync_copy(dst_hbm.at[pl.ds(cb, _DCH)], dv)

        def sub_body(s, c2):
            posv, totv = c2
            d = dv[pl.ds(s * 16, 16)]
            dl = d - lo
            mask = (dl >= 0) & (dl < _NPT)
            eid = cb + s * 16 + iota
            packed = (eid << 9) | jnp.where(mask, dl, 0)
            _, sortedv, _ = plsc.sort_key_val(iota, packed, mask=mask)
            plsc.store_scatter(buf, [posv + iota], sortedv)
            c16 = plsc.all_reduce_population_count(mask)
            return posv + c16, totv + c16

        posv0 = jnp.full((16,), pos, jnp.int32)
        posv, tot_vec = lax.fori_loop(0, _DCH // 16, sub_body, (posv0, tot_vec))
        pos = posv[0]

        def flush(args):
            pos3, fl3 = args
            dst_off = pl.multiple_of(pbase + fl3 * _PBLK, 8)
            pltpu.sync_copy(buf.at[pl.ds(0, _PBLK)], p_hbm.at[pl.ds(dst_off, _PBLK)])

            def mv(r, c4):
                buf[pl.ds(r * 16, 16)] = buf[pl.ds(_PBLK + r * 16, 16)]
                return c4

            lax.fori_loop(0, _PBLK // 16, mv, 0)
            return pos3 - _PBLK, fl3 + 1

        pos, fl = lax.cond(pos >= _PBLK, flush, lambda a: a, (pos, fl))
        return pos, fl, tot_vec

    zero = jnp.zeros((), jnp.int32)
    pos, fl, tot_vec = lax.fori_loop(
        0, N_EDGES // _DCH, chunk_body, (zero, zero, jnp.zeros((16,), jnp.int32)))
    dst_off = pl.multiple_of(pbase + fl * _PBLK, 8)
    pltpu.sync_copy(buf.at[pl.ds(0, _PBLK)], p_hbm.at[pl.ds(dst_off, _PBLK)])
    cv[...] = tot_vec
    pltpu.sync_copy(cv, c_hbm.at[pl.ds(wid * 16, 16)])


_sc_pack = functools.partial(
    pl.kernel,
    mesh=plsc.VectorSubcoreMesh(core_axis_name="c", subcore_axis_name="s"),
    compiler_params=pltpu.CompilerParams(needs_layout_passes=False),
    out_type=[
        jax.ShapeDtypeStruct((_NW * _PCAP,), jnp.int32),
        jax.ShapeDtypeStruct((_NW * 16,), jnp.int32),
    ],
    scratch_types=[
        pltpu.VMEM((_DCH,), jnp.int32),
        pltpu.VMEM((2 * _PBLK + 16,), jnp.int32),
        pltpu.VMEM((16,), jnp.int32),
        pltpu.SemaphoreType.DMA,
    ],
)(_sc_pack_body)


def _sc_scatter_body(p_hbm, c_hbm, m_hbm, agg_hbm,
                     packed_v, ridx, rows_v, cv, acc, sem):
    wid = lax.axis_index("s") * _SC_NC + lax.axis_index("c")
    iota = _iota16()

    def zero(r, c):
        for j in range(8):
            acc[r, pl.ds(16 * j, 16)] = jnp.zeros((16,), jnp.float32)
        return c

    lax.fori_loop(0, _NPT // 2, zero, 0)

    pltpu.sync_copy(c_hbm.at[pl.ds(wid * 16, 16)], cv)
    cnt = cv[...][0]
    nchunks = (cnt + _SCH - 1) // _SCH

    def chunk(k, carry):
        base = pl.multiple_of(wid * _PCAP + k * _SCH, 8)
        pltpu.sync_copy(p_hbm.at[pl.ds(base, _SCH)], packed_v)

        def mkidx(s, c2):
            pk = packed_v[pl.ds(s * 16, 16)]
            valid = (k * _SCH + s * 16 + iota) < cnt
            eidv = pk >> 9
            rowv = eidv - jnp.where(eidv >= N_EDGES // 2, N_EDGES // 2, 0)
            ridx[pl.ds(s * 16, 16)] = jnp.where(valid, rowv, 0)
            return c2

        lax.fori_loop(0, _SCH // 16, mkidx, 0)
        pltpu.async_copy(m_hbm.at[ridx], rows_v, sem).wait()
        nedge = jnp.minimum(_SCH, cnt - k * _SCH)

        def edge(i, c3):
            isp = jnp.full((16,), i, jnp.int32)
            pk = plsc.load_gather(packed_v, [isp])
            dl = pk & 511
            eid = pk >> 9
            arow = dl >> 1
            acol0 = (dl & 1) * 64
            mcol0 = jnp.where(eid >= N_EDGES // 2, 64, 0)
            for j in range(4):
                mcol = mcol0 + 16 * j + iota
                acol = acol0 + 16 * j + iota
                mv = plsc.load_gather(rows_v, [isp, mcol])
                av = plsc.load_gather(acc, [arow, acol])
                plsc.store_scatter(acc, [arow, acol], jnp.maximum(av, mv))
            return c3

        lax.fori_loop(0, nedge, edge, 0)
        return carry

    lax.fori_loop(0, nchunks, chunk, 0)
    obase = pl.multiple_of(wid * (_NPT // 2), 8)
    pltpu.sync_copy(acc, agg_hbm.at[pl.ds(obase, _NPT // 2)])


_sc_scatter = functools.partial(
    pl.kernel,
    mesh=plsc.VectorSubcoreMesh(core_axis_name="c", subcore_axis_name="s"),
    compiler_params=pltpu.CompilerParams(needs_layout_passes=False),
    out_type=jax.ShapeDtypeStruct((_NW * _NPT // 2, 2 * HID), jnp.float32),
    scratch_types=[
        pltpu.VMEM((_SCH,), jnp.int32),
        pltpu.VMEM((_SCH,), jnp.int32),
        pltpu.VMEM((_SCH, 2 * HID), jnp.float32),
        pltpu.VMEM((16,), jnp.int32),
        pltpu.VMEM((_NPT // 2, 2 * HID), jnp.float32),
        pltpu.SemaphoreType.DMA,
    ],
)(_sc_scatter_body)


def _conv_block(h, batch_3d, src, dst, p_arr, c_arr, gn, mlp, blk):
    d = h.shape[1]
    s1, s2, cnt = _graph_stats(h, batch_3d, blk)
    cnt1 = jnp.maximum(cnt[:, :1], 1.0)
    mean = s1 / cnt1
    ms = gn['mean_scale'][None, :]
    var = s2 / cnt1 - (2.0 * ms - ms * ms) * mean * mean
    w1 = mlp['W1']
    w1a = w1[:, :d]
    w1b = w1[:, d:]
    wcat = jnp.concatenate([(w1a - w1b).T, w1b.T], axis=1)
    bcat = jnp.concatenate([mlp['b1'], jnp.zeros((HID,), jnp.float32)]).reshape(1, 2 * HID)
    t = _norm_proj(h, batch_3d, mean, var, gn, wcat, bcat, blk)
    gd, gs = _sc_gather(t, src, dst)
    z = jnp.zeros((HID, HID), jnp.float32)
    w2d = jnp.block([[mlp['W2'].T, z], [z, mlp['W2'].T]])
    w3d = jnp.block([[mlp['W3'].T, z], [z, mlp['W3'].T]])
    b2d = jnp.tile(mlp['b2'], 2).reshape(1, 2 * HID)
    b3d = jnp.tile(mlp['b3'], 2).reshape(1, 2 * HID)
    m = _edge_mlp(gd, gs, w2d, b2d, w3d, b3d, 1280)
    agg = _sc_scatter(p_arr, c_arr, m)
    return agg.reshape(_NW * _NPT, HID)[:N_NODES]


def kernel(x, params, edge_index, batch):
    src = edge_index[0]
    dst = edge_index[1]
    batch_3d = batch.reshape(N_NODES // 2000, 1, 2000)
    p_arr, c_arr = _sc_pack(dst)

    h = _conv_block(x, batch_3d, src, dst, p_arr, c_arr, params['gn0'], params['conv1'], 2000)
    h = _conv_block(h, batch_3d, src, dst, p_arr, c_arr, params['gn1'], params['conv2'], 2000)
    h = _conv_block(h, batch_3d, src, dst, p_arr, c_arr, params['gn2'], params['conv3'], 2000)

    s1, _, cnt = _graph_stats(h, batch_3d, 2000)
    return _head(s1, cnt, params['dense'], params['output'])


# speedup vs baseline: 2.2917x; 1.0095x over previous
import functools
import math

import jax
import jax.numpy as jnp
from jax import lax
from jax.experimental import pallas as pl
from jax.experimental.pallas import tpu as pltpu
from jax.experimental.pallas import tpu_sc as plsc

N_NODES = 10000
N_EDGES = 320000
D_IN = 128
HID = 64
N_CLASSES = 2
N_GRAPHS = 100
EPS = 1e-5

_SELU_ALPHA = 1.6732632423543772
_SELU_SCALE = 1.0507009873554805

_INTERPRET = False


def _selu(x):
    return _SELU_SCALE * jnp.where(x > 0, x, _SELU_ALPHA * (jnp.exp(x) - 1.0))


def _stats_body(x_ref, b_ref, s1_ref, s2_ref, cnt_ref):
    i = pl.program_id(0)
    x = x_ref[...]
    batch = b_ref[0, 0]
    onehot = (batch[:, None] == jax.lax.broadcasted_iota(jnp.int32, (1, N_GRAPHS), 1)).astype(jnp.float32)

    @pl.when(i == 0)
    def _init():
        s1_ref[...] = jnp.zeros_like(s1_ref)
        s2_ref[...] = jnp.zeros_like(s2_ref)
        cnt_ref[...] = jnp.zeros_like(cnt_ref)

    s1_ref[...] += jnp.dot(onehot.T, x, preferred_element_type=jnp.float32)
    s2_ref[...] += jnp.dot(onehot.T, x * x, preferred_element_type=jnp.float32)
    cnt_ref[...] += jnp.sum(onehot, axis=0)[:, None]


def _graph_stats(x, batch_3d, blk):
    n, d = x.shape
    grid = n // blk
    return pl.pallas_call(
        _stats_body,
        grid=(grid,),
        in_specs=[
            pl.BlockSpec((blk, d), lambda i: (i, 0)),
            pl.BlockSpec((1, 1, blk), lambda i: (i, 0, 0)),
        ],
        out_specs=[
            pl.BlockSpec((N_GRAPHS, d), lambda i: (0, 0)),
            pl.BlockSpec((N_GRAPHS, d), lambda i: (0, 0)),
            pl.BlockSpec((N_GRAPHS, 128), lambda i: (0, 0)),
        ],
        out_shape=[
            jax.ShapeDtypeStruct((N_GRAPHS, d), jnp.float32),
            jax.ShapeDtypeStruct((N_GRAPHS, d), jnp.float32),
            jax.ShapeDtypeStruct((N_GRAPHS, 128), jnp.float32),
        ],
        interpret=_INTERPRET,
    )(x, batch_3d)


def _norm_proj_body(x_ref, b_ref, mean_ref, var_ref, gw_ref, gb_ref, gms_ref,
                    wcat_ref, bcat_ref, t_ref):
    x = x_ref[...]
    batch = b_ref[0, 0]
    onehot = (batch[:, None] == jax.lax.broadcasted_iota(jnp.int32, (1, N_GRAPHS), 1)).astype(jnp.float32)
    mean = jnp.dot(onehot, mean_ref[...], preferred_element_type=jnp.float32)
    var = jnp.dot(onehot, var_ref[...], preferred_element_type=jnp.float32)
    out = x - gms_ref[...] * mean
    nrm = gw_ref[...] * (out * jax.lax.rsqrt(var + EPS)) + gb_ref[...]
    t_ref[...] = jnp.dot(nrm, wcat_ref[...], preferred_element_type=jnp.float32) + bcat_ref[...]


def _norm_proj(x, batch_3d, mean, var, gn, wcat, bcat, blk):
    n, d = x.shape
    grid = n // blk
    gw = gn['weight'].reshape(1, d)
    gb = gn['bias'].reshape(1, d)
    gms = gn['mean_scale'].reshape(1, d)
    return pl.pallas_call(
        _norm_proj_body,
        grid=(grid,),
        in_specs=[
            pl.BlockSpec((blk, d), lambda i: (i, 0)),
            pl.BlockSpec((1, 1, blk), lambda i: (i, 0, 0)),
            pl.BlockSpec((N_GRAPHS, d), lambda i: (0, 0)),
            pl.BlockSpec((N_GRAPHS, d), lambda i: (0, 0)),
            pl.BlockSpec((1, d), lambda i: (0, 0)),
            pl.BlockSpec((1, d), lambda i: (0, 0)),
            pl.BlockSpec((1, d), lambda i: (0, 0)),
            pl.BlockSpec((d, 2 * HID), lambda i: (0, 0)),
            pl.BlockSpec((1, 2 * HID), lambda i: (0, 0)),
        ],
        out_specs=pl.BlockSpec((blk, 2 * HID), lambda i: (i, 0)),
        out_shape=jax.ShapeDtypeStruct((n, 2 * HID), jnp.float32),
        interpret=_INTERPRET,
    )(x, batch_3d, mean, var, gw, gb, gms, wcat, bcat)


def _edge_mlp_body(gd1_ref, gs1_ref, gd2_ref, gs2_ref,
                   w2_ref, b2_ref, w3_ref, b3_ref, out_ref):
    e1 = gd1_ref[...][:, :HID] + gs1_ref[...][:, HID:]
    e2 = gd2_ref[...][:, :HID] + gs2_ref[...][:, HID:]
    m = _selu(jnp.concatenate([e1, e2], axis=1))
    m = _selu(jnp.dot(m, w2_ref[...], preferred_element_type=jnp.float32) + b2_ref[...])
    out_ref[...] = jnp.dot(m, w3_ref[...], preferred_element_type=jnp.float32) + b3_ref[...]


def _edge_mlp(gd, gs, w2d, b2d, w3d, b3d, blk):
    h2 = 2 * HID
    half_blocks = (N_EDGES // 2) // blk
    grid = half_blocks
    return pl.pallas_call(
        _edge_mlp_body,
        grid=(grid,),
        in_specs=[
            pl.BlockSpec((blk, h2), lambda i: (i, 0)),
            pl.BlockSpec((blk, h2), lambda i: (i, 0)),
            pl.BlockSpec((blk, h2), lambda i: (i + half_blocks, 0)),
            pl.BlockSpec((blk, h2), lambda i: (i + half_blocks, 0)),
            pl.BlockSpec((h2, h2), lambda i: (0, 0)),
            pl.BlockSpec((1, h2), lambda i: (0, 0)),
            pl.BlockSpec((h2, h2), lambda i: (0, 0)),
            pl.BlockSpec((1, h2), lambda i: (0, 0)),
        ],
        out_specs=pl.BlockSpec((blk, h2), lambda i: (i, 0)),
        out_shape=jax.ShapeDtypeStruct((N_EDGES // 2, h2), jnp.float32),
        interpret=_INTERPRET,
    )(gd, gs, gd, gs, w2d, b2d, w3d, b3d)


def _head_body(s1_ref, cnt_ref, wd_ref, bd_ref, wo_ref, bo_ref, out_ref):
    cnt = jnp.maximum(cnt_ref[...][:, :1], 1.0)
    pooled = s1_ref[...] / cnt
    h = jnp.maximum(jnp.dot(pooled, wd_ref[...], preferred_element_type=jnp.float32) + bd_ref[...], 0.0)
    logits = jnp.dot(h, wo_ref[...], preferred_element_type=jnp.float32) + bo_ref[...]
    mx = jnp.max(logits, axis=1, keepdims=True)
    ex = jnp.exp(logits - mx)
    out_ref[...] = ex / jnp.sum(ex, axis=1, keepdims=True)


def _head(s1, cnt, dense, output):
    wd_t = dense['W'].T
    wo_t = output['W'].T
    return pl.pallas_call(
        _head_body,
        out_shape=jax.ShapeDtypeStruct((N_GRAPHS, N_CLASSES), jnp.float32),
        interpret=_INTERPRET,
    )(s1, cnt, wd_t, dense['b'].reshape(1, HID), wo_t, output['b'].reshape(1, N_CLASSES))


_SC_NC = 2
_SC_NS = 16
_NW = _SC_NC * _SC_NS
_EPW = N_EDGES // _NW
_ECHUNK = 400
_L = 16


def _sc_gather_body(t_hbm, src_hbm, dst_hbm, gd_hbm, gs_hbm,
                    dstv, srcv, rowsd, rowss, sem):
    wid = lax.axis_index("s") * _SC_NC + lax.axis_index("c")
    base0 = pl.multiple_of(wid * _EPW, 8)
    ci = pltpu.async_copy(dst_hbm.at[pl.ds(base0, _EPW)], dstv, sem)
    cj = pltpu.async_copy(src_hbm.at[pl.ds(base0, _EPW)], srcv, sem)
    ci.wait()
    cj.wait()

    def body(i, carry):
        off = i * _ECHUNK
        base = pl.multiple_of(base0 + off, 8)
        cd = pltpu.async_copy(t_hbm.at[dstv.at[pl.ds(off, _ECHUNK)]], rowsd, sem)
        cs = pltpu.async_copy(t_hbm.at[srcv.at[pl.ds(off, _ECHUNK)]], rowss, sem)
        cd.wait()
        cs.wait()
        co = pltpu.async_copy(rowsd, gd_hbm.at[pl.ds(base, _ECHUNK)], sem)
        cp = pltpu.async_copy(rowss, gs_hbm.at[pl.ds(base, _ECHUNK)], sem)
        co.wait()
        cp.wait()
        return carry

    lax.fori_loop(0, _EPW // _ECHUNK, body, 0)


_sc_gather = functools.partial(
    pl.kernel,
    mesh=plsc.VectorSubcoreMesh(core_axis_name="c", subcore_axis_name="s"),
    out_type=[
        jax.ShapeDtypeStruct((N_EDGES, 2 * HID), jnp.float32),
        jax.ShapeDtypeStruct((N_EDGES, 2 * HID), jnp.float32),
    ],
    scratch_types=[
        pltpu.VMEM((_EPW,), jnp.int32),
        pltpu.VMEM((_EPW,), jnp.int32),
        pltpu.VMEM((_ECHUNK, 2 * HID), jnp.float32),
        pltpu.VMEM((_ECHUNK, 2 * HID), jnp.float32),
        pltpu.SemaphoreType.DMA,
    ],
)(_sc_gather_body)


_NPT = 320
_PCAP = N_EDGES + 2048
_PBLK = 2048
_DCH = 2048
_SCH = 512


def _iota16():
    return lax.iota(jnp.int32, 16)


def _sc_pack_body(dst_hbm, p_hbm, c_hbm, dv, buf, cv, sem):
    wid = lax.axis_index("s") * _SC_NC + lax.axis_index("c")
    lo = wid * _NPT
    pbase = wid * _PCAP
    iota = _iota16()

    def chunk_body(ci, carry):
        pos, fl, tot_vec = carry
        cb = ci * _DCH
        pltpu.sync_copy(dst_hbm.at[pl.ds(cb, _DCH)], dv)

        def sub_body(s, c2):
            posv, totv = c2
            for u in range(4):
                d = dv[pl.ds((s * 4 + u) * 16, 16)]
                dl = d - lo
                mask = (dl >= 0) & (dl < _NPT)
                eid = cb + (s * 4 + u) * 16 + iota
                packed = (eid << 9) | jnp.where(mask, dl, 0)
                _, sortedv, _ = plsc.sort_key_val(iota, packed, mask=mask)
                plsc.store_scatter(buf, [posv + iota], sortedv)
                c16 = plsc.all_reduce_population_count(mask)
                posv = posv + c16
                totv = totv + c16
            return posv, totv

        posv0 = jnp.full((16,), pos, jnp.int32)
        posv, tot_vec = lax.fori_loop(0, _DCH // 64, sub_body, (posv0, tot_vec))
        pos = posv[0]

        def flush(args):
            pos3, fl3 = args
            dst_off = pl.multiple_of(pbase + fl3 * _PBLK, 8)
            pltpu.sync_copy(buf.at[pl.ds(0, _PBLK)], p_hbm.at[pl.ds(dst_off, _PBLK)])

            def mv(r, c4):
                buf[pl.ds(r * 16, 16)] = buf[pl.ds(_PBLK + r * 16, 16)]
                return c4

            lax.fori_loop(0, _PBLK // 16, mv, 0)
            return pos3 - _PBLK, fl3 + 1

        pos, fl = lax.cond(pos >= _PBLK, flush, lambda a: a, (pos, fl))
        return pos, fl, tot_vec

    zero = jnp.zeros((), jnp.int32)
    pos, fl, tot_vec = lax.fori_loop(
        0, N_EDGES // _DCH, chunk_body, (zero, zero, jnp.zeros((16,), jnp.int32)))
    dst_off = pl.multiple_of(pbase + fl * _PBLK, 8)
    pltpu.sync_copy(buf.at[pl.ds(0, _PBLK)], p_hbm.at[pl.ds(dst_off, _PBLK)])
    cv[...] = tot_vec
    pltpu.sync_copy(cv, c_hbm.at[pl.ds(wid * 16, 16)])


_sc_pack = functools.partial(
    pl.kernel,
    mesh=plsc.VectorSubcoreMesh(core_axis_name="c", subcore_axis_name="s"),
    compiler_params=pltpu.CompilerParams(needs_layout_passes=False),
    out_type=[
        jax.ShapeDtypeStruct((_NW * _PCAP,), jnp.int32),
        jax.ShapeDtypeStruct((_NW * 16,), jnp.int32),
    ],
    scratch_types=[
        pltpu.VMEM((_DCH,), jnp.int32),
        pltpu.VMEM((2 * _PBLK + 16,), jnp.int32),
        pltpu.VMEM((16,), jnp.int32),
        pltpu.SemaphoreType.DMA,
    ],
)(_sc_pack_body)


def _sc_scatter_body(p_hbm, c_hbm, m_hbm, agg_hbm,
                     packed_v, ridx, rows_v, cv, acc, sem):
    wid = lax.axis_index("s") * _SC_NC + lax.axis_index("c")
    iota = _iota16()

    def zero(r, c):
        for j in range(8):
            acc[r, pl.ds(16 * j, 16)] = jnp.zeros((16,), jnp.float32)
        return c

    lax.fori_loop(0, _NPT // 2, zero, 0)

    pltpu.sync_copy(c_hbm.at[pl.ds(wid * 16, 16)], cv)
    cnt = cv[...][0]
    nchunks = (cnt + _SCH - 1) // _SCH

    def chunk(k, carry):
        base = pl.multiple_of(wid * _PCAP + k * _SCH, 8)
        pltpu.sync_copy(p_hbm.at[pl.ds(base, _SCH)], packed_v)

        def mkidx(s, c2):
            pk = packed_v[pl.ds(s * 16, 16)]
            valid = (k * _SCH + s * 16 + iota) < cnt
            eidv = pk >> 9
            rowv = eidv - jnp.where(eidv >= N_EDGES // 2, N_EDGES // 2, 0)
            ridx[pl.ds(s * 16, 16)] = jnp.where(valid, rowv, 0)
            return c2

        lax.fori_loop(0, _SCH // 16, mkidx, 0)
        pltpu.async_copy(m_hbm.at[ridx], rows_v, sem).wait()
        nedge = jnp.minimum(_SCH, cnt - k * _SCH)

        def edge(i, c3):
            isp = jnp.full((16,), i, jnp.int32)
            pk = plsc.load_gather(packed_v, [isp])
            dl = pk & 511
            eid = pk >> 9
            arow = dl >> 1
            acol0 = (dl & 1) * 64
            mcol0 = jnp.where(eid >= N_EDGES // 2, 64, 0)
            for j in range(4):
                mcol = mcol0 + 16 * j + iota
                acol = acol0 + 16 * j + iota
                mv = plsc.load_gather(rows_v, [isp, mcol])
                av = plsc.load_gather(acc, [arow, acol])
                plsc.store_scatter(acc, [arow, acol], jnp.maximum(av, mv))
            return c3

        lax.fori_loop(0, nedge, edge, 0)
        return carry

    lax.fori_loop(0, nchunks, chunk, 0)
    obase = pl.multiple_of(wid * (_NPT // 2), 8)
    pltpu.sync_copy(acc, agg_hbm.at[pl.ds(obase, _NPT // 2)])


_sc_scatter = functools.partial(
    pl.kernel,
    mesh=plsc.VectorSubcoreMesh(core_axis_name="c", subcore_axis_name="s"),
    compiler_params=pltpu.CompilerParams(needs_layout_passes=False),
    out_type=jax.ShapeDtypeStruct((_NW * _NPT // 2, 2 * HID), jnp.float32),
    scratch_types=[
        pltpu.VMEM((_SCH,), jnp.int32),
        pltpu.VMEM((_SCH,), jnp.int32),
        pltpu.VMEM((_SCH, 2 * HID), jnp.float32),
        pltpu.VMEM((16,), jnp.int32),
        pltpu.VMEM((_NPT // 2, 2 * HID), jnp.float32),
        pltpu.SemaphoreType.DMA,
    ],
)(_sc_scatter_body)


def _conv_block(h, batch_3d, src, dst, p_arr, c_arr, gn, mlp, blk):
    d = h.shape[1]
    s1, s2, cnt = _graph_stats(h, batch_3d, blk)
    cnt1 = jnp.maximum(cnt[:, :1], 1.0)
    mean = s1 / cnt1
    ms = gn['mean_scale'][None, :]
    var = s2 / cnt1 - (2.0 * ms - ms * ms) * mean * mean
    w1 = mlp['W1']
    w1a = w1[:, :d]
    w1b = w1[:, d:]
    wcat = jnp.concatenate([(w1a - w1b).T, w1b.T], axis=1)
    bcat = jnp.concatenate([mlp['b1'], jnp.zeros((HID,), jnp.float32)]).reshape(1, 2 * HID)
    t = _norm_proj(h, batch_3d, mean, var, gn, wcat, bcat, blk)
    gd, gs = _sc_gather(t, src, dst)
    z = jnp.zeros((HID, HID), jnp.float32)
    w2d = jnp.block([[mlp['W2'].T, z], [z, mlp['W2'].T]])
    w3d = jnp.block([[mlp['W3'].T, z], [z, mlp['W3'].T]])
    b2d = jnp.tile(mlp['b2'], 2).reshape(1, 2 * HID)
    b3d = jnp.tile(mlp['b3'], 2).reshape(1, 2 * HID)
    m = _edge_mlp(gd, gs, w2d, b2d, w3d, b3d, 1280)
    agg = _sc_scatter(p_arr, c_arr, m)
    return agg.reshape(_NW * _NPT, HID)[:N_NODES]


def kernel(x, params, edge_index, batch):
    src = edge_index[0]
    dst = edge_index[1]
    batch_3d = batch.reshape(N_NODES // 2000, 1, 2000)
    p_arr, c_arr = _sc_pack(dst)

    h = _conv_block(x, batch_3d, src, dst, p_arr, c_arr, params['gn0'], params['conv1'], 2000)
    h = _conv_block(h, batch_3d, src, dst, p_arr, c_arr, params['gn1'], params['conv2'], 2000)
    h = _conv_block(h, batch_3d, src, dst, p_arr, c_arr, params['gn2'], params['conv3'], 2000)

    s1, _, cnt = _graph_stats(h, batch_3d, 2000)
    return _head(s1, cnt, params['dense'], params['output'])
